# Initial kernel scaffold; baseline (speedup 1.0000x reference)
import jax
import jax.numpy as jnp
from jax.experimental import pallas as pl


def kernel(corners3d, neck_voxel_sizes):
    raise NotImplementedError("write your pallas kernel here")


# speedup vs baseline: 2.8560x; 2.8560x over previous
import functools

import jax
import jax.numpy as jnp
from jax import lax
from jax.experimental import pallas as pl
from jax.experimental.pallas import tpu as pltpu
from jax.experimental.pallas import tpu_sc as plsc

CUBE = 32
KPAD = 48
LOW = -(CUBE // 2)
NBOX = 256
G = CUBE * CUBE * CUBE
F32 = jnp.float32
I32 = jnp.int32


def _tables_body(cx_ref, cy_ref, cz_ref, czt_ref, vox_ref,
                 ox_ref, oy_ref, ozt_ref, aux_ref):
    cx = cx_ref[:]
    cy = cy_ref[:]
    cz = cz_ref[:]
    czt = czt_ref[:]
    vx = vox_ref[0:1, 0:1]
    vy = vox_ref[1:2, 0:1]
    vz = vox_ref[2:3, 0:1]

    bminx = jnp.min(cx, axis=0, keepdims=True)
    bmaxx = jnp.max(cx, axis=0, keepdims=True)
    bminy = jnp.min(cy, axis=0, keepdims=True)
    bmaxy = jnp.max(cy, axis=0, keepdims=True)
    bminz = jnp.min(cz, axis=0, keepdims=True)
    bmaxz = jnp.max(cz, axis=0, keepdims=True)

    bvol = ((bmaxx - bminx) * (bmaxy - bminy)) * (bmaxz - bminz)

    hy = cy[0:1, :] - cy[3:4, :]
    hx = cx[0:1, :] - cx[3:4, :]
    ang = jnp.arctan2(hy, hx)
    sinr = jnp.sin(ang)
    cosr = jnp.cos(ang)

    maxh = jnp.max(cz)
    minh = jnp.min(cz)

    iv = lax.broadcasted_iota(I32, (CUBE, NBOX), 0).astype(F32) + LOW
    gx0 = iv * vx
    gx1 = (iv + 1.0) * vx
    ox_ref[:] = jnp.maximum(jnp.minimum(gx1, bmaxx) - jnp.maximum(gx0, bminx), 0.0)
    gy0 = iv * vy
    gy1 = (iv + 1.0) * vy
    oy_ref[:] = jnp.maximum(jnp.minimum(gy1, bmaxy) - jnp.maximum(gy0, bminy), 0.0)

    kvi = lax.broadcasted_iota(I32, (NBOX, KPAD), 1)
    kvt = kvi.astype(F32) + LOW
    bminzt = jnp.min(czt, axis=1, keepdims=True)
    bmaxzt = jnp.max(czt, axis=1, keepdims=True)
    gz0 = kvt * vz
    gz1 = (kvt + 1.0) * vz
    ozt = jnp.maximum(jnp.minimum(gz1, bmaxzt) - jnp.maximum(gz0, bminzt), 0.0)
    ozt = jnp.where(kvi < CUBE, ozt, 0.0)
    ozt_ref[:] = ozt

    ozmax = jnp.max(ozt, axis=0, keepdims=True)
    kact = ozmax > 0.0
    klane = lax.broadcasted_iota(I32, (1, KPAD), 1)
    klo = jnp.min(jnp.where(kact, klane, 9999))
    khi = jnp.max(jnp.where(kact, klane, -1))
    nkg = jnp.maximum((khi - klo + 16) // 16, 0)

    lane_i = lax.broadcasted_iota(I32, (1, NBOX), 1)
    lane = lane_i.astype(F32)
    c0 = lane + LOW
    c1 = c0 + 1.0
    dxr = c1 * vx - c0 * vx
    dyr = c1 * vy - c0 * vy
    dzr = jnp.where(lane_i < CUBE, c1 * vz - c0 * vz, 0.0)
    z0 = c0 * vz
    z1 = c1 * vz
    outside = (z0 > maxh) | (z1 < minh)
    insider = jnp.where(outside | (lane_i >= CUBE), 0.0, 1.0)
    voxvol = (vx * vy) * vz
    thresh = jnp.broadcast_to(0.5 * voxvol, (1, NBOX))
    constr = jnp.where(lane_i == 1, klo.astype(F32),
                       jnp.where(lane_i == 2, nkg.astype(F32), 0.0))
    constr = jnp.where(lane_i == 0, thresh, constr)
    aux_ref[:] = jnp.concatenate(
        [bvol, sinr, cosr, dxr, dyr, dzr, insider, constr], axis=0)


def _sweep_body(oxf_hbm, oyf_hbm, oztf_hbm, aux_hbm, out_hbm,
                oxf_v, oyf_v, oztf_v, bvol_v, sin_v, cos_v,
                dx_v, dy_v, dz_v, ins_v, th_v, act_v, res_v, dsem):
    c = lax.axis_index("c")
    s = lax.axis_index("s")
    wid = s * 2 + c

    pltpu.sync_copy(oxf_hbm, oxf_v)
    pltpu.sync_copy(oyf_hbm, oyf_v)
    pltpu.sync_copy(oztf_hbm, oztf_v)
    pltpu.sync_copy(aux_hbm.at[0], bvol_v)
    pltpu.sync_copy(aux_hbm.at[1], sin_v)
    pltpu.sync_copy(aux_hbm.at[2], cos_v)
    pltpu.sync_copy(aux_hbm.at[3, pl.ds(0, 32)], dx_v)
    pltpu.sync_copy(aux_hbm.at[4, pl.ds(0, 32)], dy_v)
    pltpu.sync_copy(aux_hbm.at[5, pl.ds(0, KPAD)], dz_v)
    pltpu.sync_copy(aux_hbm.at[6, pl.ds(0, KPAD)], ins_v)
    pltpu.sync_copy(aux_hbm.at[7, pl.ds(0, 16)], th_v)

    lanes = lax.iota(I32, 16)
    thv = th_v[...]
    thresh = thv[0]
    klo = thv[1].astype(I32)
    nkg = thv[2].astype(I32)
    zero = jnp.zeros((16,), F32)

    def q_body(q, carry):
        i = wid + q
        i = i - 32 * (i >= 32).astype(I32)
        j = q
        i_s = jnp.full((16,), 0, I32) + i
        j_s = jnp.full((16,), 0, I32) + j
        dxi = plsc.load_gather(dx_v, [i_s])
        dyj = plsc.load_gather(dy_v, [j_s])
        gxy = dxi * dyj
        irow = i * NBOX
        jrow = j * NBOX

        for cz in range(4):
            res_v[pl.ds(q * 64 + cz * 16, 16)] = zero

        cnt = jnp.int32(0)
        for cch in range(NBOX // 16):
            base = cch * 16
            sxyv = oxf_v[pl.ds(irow + base, 16)] * oyf_v[pl.ds(jrow + base, 16)]
            m = sxyv > 0.0
            plsc.store_compressed(act_v.at[pl.ds(cnt, 16)], base + lanes,
                                  mask=m)
            cnt = cnt + plsc.all_reduce_population_count(m)[0]
        nch = (cnt + 15) // 16

        def kg_body(kg, carry2):
            koff = klo + kg * 16
            kvec = koff + lanes
            inb = kvec < CUBE
            dzv = plsc.load_gather(dz_v, [kvec])
            insv = plsc.load_gather(ins_v, [kvec])
            gvolv = gxy * dzv

            def c_body(cb, st):
                bv, bi = st
                idxr = act_v[pl.ds(cb * 16, 16)]
                valid = (cb * 16 + lanes) < cnt
                idxv = jnp.where(valid, idxr, 0)
                oxg = plsc.load_gather(oxf_v, [irow + idxv])
                oyg = plsc.load_gather(oyf_v, [jrow + idxv])
                sxyv = jnp.where(valid, oxg * oyg, 0.0)
                bvv = plsc.load_gather(bvol_v, [idxv])
                ozbase = idxv * KPAD + koff
                for t in range(16):
                    ozidx = jnp.full((16,), 0, I32) + ozbase[t] + lanes
                    ozv = plsc.load_gather(oztf_v, [ozidx])
                    inter = sxyv[t] * ozv
                    u = (gvolv + bvv[t]) - inter
                    iou = inter / jnp.maximum(u, F32(0.1))
                    upd = iou > bv
                    bv = jnp.where(upd, iou, bv)
                    bi = jnp.where(upd, idxv[t], bi)
                return bv, bi

            init = (jnp.full((16,), -1.0, F32),
                    jnp.zeros((16,), I32))
            bv, bi = lax.fori_loop(0, nch, c_body, init)

            oxg2 = plsc.load_gather(oxf_v, [irow + bi])
            oyg2 = plsc.load_gather(oyf_v, [jrow + bi])
            ozg2 = plsc.load_gather(oztf_v, [bi * KPAD + koff + lanes])
            bint = (oxg2 * oyg2) * ozg2

            ok = (bint > thresh) & (insv > 0.5)
            sinv = plsc.load_gather(sin_v, [bi])
            cosv = plsc.load_gather(cos_v, [bi])
            cs = jnp.where(ok, sinv, zero)
            cc = jnp.where(ok, cosv, zero)
            ridx = q * 64 + kvec * 2
            plsc.store_scatter(res_v, [ridx], cs, mask=inb)
            plsc.store_scatter(res_v, [ridx + 1], cc, mask=inb)
            return carry2

        lax.fori_loop(0, nkg, kg_body, 0)

        pltpu.async_copy(res_v.at[pl.ds(q * 64, 64)],
                         out_hbm.at[pl.ds((i * 1024 + j * 32) * 2, 64)], dsem)
        return carry

    lax.fori_loop(0, CUBE, q_body, 0)

    def drain_body(q, carry):
        pltpu.make_async_copy(
            res_v.at[pl.ds(0, 64)],
            out_hbm.at[pl.ds(0, 64)], dsem).wait()
        return carry

    lax.fori_loop(0, CUBE, drain_body, 0)


def kernel(corners3d, neck_voxel_sizes):
    corners3d = corners3d.astype(F32)
    cx = corners3d[:, :, 0].T
    cy = corners3d[:, :, 1].T
    cz = corners3d[:, :, 2].T
    czt = corners3d[:, :, 2]
    vox = jnp.broadcast_to(neck_voxel_sizes.astype(F32).reshape(3, 1), (3, 128))
    vox = jnp.concatenate([vox, jnp.zeros((5, 128), F32)], axis=0)

    ox, oy, ozt, aux = pl.pallas_call(
        _tables_body,
        out_shape=[
            jax.ShapeDtypeStruct((CUBE, NBOX), F32),
            jax.ShapeDtypeStruct((CUBE, NBOX), F32),
            jax.ShapeDtypeStruct((NBOX, KPAD), F32),
            jax.ShapeDtypeStruct((8, NBOX), F32),
        ],
    )(cx, cy, cz, czt, vox)

    mesh = plsc.VectorSubcoreMesh(core_axis_name="c", subcore_axis_name="s",
                                  num_cores=2, num_subcores=16)
    sweep = functools.partial(
        pl.kernel,
        out_type=jax.ShapeDtypeStruct((G * 2,), F32),
        mesh=mesh,
        compiler_params=pltpu.CompilerParams(
            use_tc_tiling_on_sc=False, needs_layout_passes=False),
        scratch_types=[
            pltpu.VMEM((CUBE * NBOX,), F32),
            pltpu.VMEM((CUBE * NBOX,), F32),
            pltpu.VMEM((NBOX * KPAD,), F32),
            pltpu.VMEM((NBOX,), F32),
            pltpu.VMEM((NBOX,), F32),
            pltpu.VMEM((NBOX,), F32),
            pltpu.VMEM((CUBE,), F32),
            pltpu.VMEM((CUBE,), F32),
            pltpu.VMEM((KPAD,), F32),
            pltpu.VMEM((KPAD,), F32),
            pltpu.VMEM((16,), F32),
            pltpu.VMEM((NBOX + 16,), I32),
            pltpu.VMEM((2048,), F32),
            pltpu.SemaphoreType.DMA,
        ],
    )(_sweep_body)
    return sweep(ox.reshape(-1), oy.reshape(-1), ozt.reshape(-1),
                 aux).reshape(G, 2)
